# Initial kernel scaffold; baseline (speedup 1.0000x reference)
import jax
import jax.numpy as jnp
from jax.experimental import pallas as pl


def kernel(edge_src_feat, edge_dst_feat, dst_deg, W_self, b_self, W_neigh, b_neigh, segment_ids, dst_max_eid, current_layer):
    raise NotImplementedError("write your pallas kernel here")


# speedup vs baseline: 3.4961x; 3.4961x over previous
import functools

import jax
import jax.numpy as jnp
from jax import lax
from jax.experimental import pallas as pl
from jax.experimental.pallas import tpu as pltpu
from jax.experimental.pallas import tpu_sc as plsc

_BA = 256
_BC = 1280
_GR = 128

_HI = lax.Precision.HIGHEST


def _cumsum_body(seg_r_ref, seg_c_ref, prev_ref, x_ref, wn_ref, out_ref, carry_ref):
    i = pl.program_id(0)

    @pl.when(i == 0)
    def _():
        carry_ref[...] = jnp.zeros_like(carry_ref)

    B = x_ref.shape[0]
    seg_j = seg_r_ref[0]
    seg_i = seg_c_ref[0]
    ii = lax.broadcasted_iota(jnp.int32, (B, B), 0)
    jj = lax.broadcasted_iota(jnp.int32, (B, B), 1)
    a = ((jj <= ii) & (seg_j >= seg_i)).astype(jnp.float32)
    h = lax.dot_general(a, x_ref[...], (((1,), (0,)), ((), ())), precision=_HI)
    mask = (seg_i == prev_ref[0, 0, 0]).astype(jnp.float32)
    h = h + mask * carry_ref[...]
    carry_ref[...] = h[B - 1:B, :]
    out_ref[...] = lax.dot_general(h, wn_ref[...], (((1,), (1,)), ((), ())),
                                   precision=_HI)


def _segcumsum_matmul(x, seg32, w_neigh):
    e, d = x.shape
    nb = e // _BA
    seg_r = seg32.reshape(nb, 1, _BA)
    seg_c = seg32.reshape(nb, _BA, 1)
    prev_seg = jnp.concatenate(
        [jnp.full((1,), -1, jnp.int32), seg32[_BA - 1::_BA][:-1]]).reshape(nb, 1, 1)
    return pl.pallas_call(
        _cumsum_body,
        grid=(nb,),
        in_specs=[
            pl.BlockSpec((1, 1, _BA), lambda i: (i, 0, 0)),
            pl.BlockSpec((1, _BA, 1), lambda i: (i, 0, 0)),
            pl.BlockSpec((1, 1, 1), lambda i: (i, 0, 0), memory_space=pltpu.SMEM),
            pl.BlockSpec((_BA, d), lambda i: (i, 0)),
            pl.BlockSpec((d, d), lambda i: (0, 0)),
        ],
        out_specs=pl.BlockSpec((_BA, d), lambda i: (i, 0)),
        out_shape=jax.ShapeDtypeStruct((e, d), jnp.float32),
        scratch_shapes=[pltpu.VMEM((1, d), jnp.float32)],
        compiler_params=pltpu.CompilerParams(
            dimension_semantics=("arbitrary",)),
    )(seg_r, seg_c, prev_seg, x, w_neigh)


def _sc_gather(g, idx2):
    e, d = g.shape
    n_ops = idx2.shape[0]
    info = plsc.get_sparse_core_info()
    nc, ns = info.num_cores, info.num_subcores
    nw = nc * ns
    iters = -(-n_ops // nw)
    mesh = plsc.VectorSubcoreMesh(core_axis_name="c", subcore_axis_name="s")

    @functools.partial(
        pl.kernel,
        out_type=jax.ShapeDtypeStruct((e, d), jnp.float32),
        mesh=mesh,
        scratch_types=[
            pltpu.VMEM((_GR,), jnp.int32),
            pltpu.VMEM((_GR, d), jnp.float32),
            pltpu.SemaphoreType.DMA,
        ],
    )
    def gather_k(g_hbm, idx_hbm, out_hbm, idx_v, rows_v, sem):
        wid = lax.axis_index("s") * nc + lax.axis_index("c")

        def step(k, c):
            op = k * nw + wid

            @pl.when(op < n_ops)
            def _():
                pltpu.sync_copy(idx_hbm.at[op], idx_v)
                pltpu.async_copy(g_hbm.at[idx_v], rows_v, sem).wait()
                pltpu.sync_copy(rows_v, out_hbm.at[pl.ds(op * _GR, _GR)])
            return c

        lax.fori_loop(0, iters, step, 0)

    return gather_k(g, idx2)


def _final_body(dst_ref, hg_ref, deg_ref, ws_ref, bs_ref, bn_ref, out_ref):
    scale = 1.0 / (deg_ref[...] + 1.0)
    t = lax.dot_general(dst_ref[...], ws_ref[...], (((1,), (1,)), ((), ())),
                        precision=_HI)
    out_ref[...] = t + hg_ref[...] * scale + bs_ref[...] + bn_ref[...]


def _final(dst, hg, deg, w_self, b_self, b_neigh):
    e, d = dst.shape
    nb = e // _BC
    return pl.pallas_call(
        _final_body,
        grid=(nb,),
        in_specs=[
            pl.BlockSpec((_BC, d), lambda i: (i, 0)),
            pl.BlockSpec((_BC, d), lambda i: (i, 0)),
            pl.BlockSpec((_BC, 1), lambda i: (i, 0)),
            pl.BlockSpec((d, d), lambda i: (0, 0)),
            pl.BlockSpec((1, d), lambda i: (0, 0)),
            pl.BlockSpec((1, d), lambda i: (0, 0)),
        ],
        out_specs=pl.BlockSpec((_BC, d), lambda i: (i, 0)),
        out_shape=jax.ShapeDtypeStruct((e, d), jnp.float32),
        compiler_params=pltpu.CompilerParams(
            dimension_semantics=("parallel",)),
    )(dst, hg, deg, w_self, b_self, b_neigh)


def kernel(edge_src_feat, edge_dst_feat, dst_deg, W_self, b_self, W_neigh,
           b_neigh, segment_ids, dst_max_eid, current_layer):
    e, d = edge_src_feat.shape
    seg32 = segment_ids.astype(jnp.int32)
    g = _segcumsum_matmul(edge_src_feat, seg32, W_neigh)
    idx2 = dst_max_eid.astype(jnp.int32).reshape(e // _GR, _GR)
    hg = _sc_gather(g, idx2)
    return _final(edge_dst_feat, hg, dst_deg.reshape(e, 1), W_self,
                  b_self.reshape(1, d), b_neigh.reshape(1, d))
